# baseline (speedup 1.0000x reference)
import functools

import jax
import jax.numpy as jnp
from jax import lax
from jax.experimental import pallas as pl
from jax.experimental.pallas import tpu as pltpu
from jax.experimental.pallas import tpu_sc as plsc


def _make_gather(n_batch: int, seq_pad: int, dim_pad: int, n_workers: int,
                 nbuf: int):
    batch_per_w = n_batch // n_workers
    idx_per_w = batch_per_w * seq_pad
    mesh = plsc.VectorSubcoreMesh(core_axis_name="c", subcore_axis_name="s")
    num_cores = mesh.num_cores

    @functools.partial(
        pl.kernel,
        out_type=jax.ShapeDtypeStruct((n_batch, seq_pad, dim_pad),
                                      jnp.float32),
        mesh=mesh,
        scratch_types=[
            pltpu.VMEM((idx_per_w,), jnp.int32),
            pltpu.VMEM((nbuf, seq_pad, dim_pad), jnp.float32),
            pltpu.SemaphoreType.DMA,
            pltpu.SemaphoreType.DMA,
        ],
    )
    def gather_kernel(table_hbm, idx_hbm, x_hbm, idx_v, rows_v, gsem, ssem):
        wid = lax.axis_index("s") * num_cores + lax.axis_index("c")
        wb = wid * batch_per_w
        pltpu.sync_copy(idx_hbm.at[pl.ds(wid * idx_per_w, idx_per_w)], idx_v)

        def start_gather(g):
            pltpu.async_copy(
                table_hbm.at[idx_v.at[pl.ds(g * seq_pad, seq_pad)]],
                rows_v.at[g % nbuf], gsem)

        def wait_gather(g):
            pltpu.make_async_copy(
                table_hbm.at[idx_v.at[pl.ds(g * seq_pad, seq_pad)]],
                rows_v.at[g % nbuf], gsem).wait()

        def start_store(g):
            pltpu.async_copy(rows_v.at[g % nbuf], x_hbm.at[wb + g], ssem)

        def wait_store(g):
            pltpu.make_async_copy(rows_v.at[g % nbuf], x_hbm.at[wb + g],
                                  ssem).wait()

        for b in range(nbuf - 1):
            start_gather(b)

        def body(g, _):
            @pl.when(g > 0)
            def _():
                wait_store(g - 1)

            @pl.when(g + nbuf - 1 < batch_per_w)
            def _():
                start_gather(g + nbuf - 1)

            wait_gather(g)
            start_store(g)
            return 0

        lax.fori_loop(0, batch_per_w, body, 0)
        wait_store(batch_per_w - 1)

    return gather_kernel


def _make_trunc(n_batch: int, seq: int, seq_pad: int, dim: int,
                dim_pad: int, ncopy: int):
    nb = n_batch // ncopy

    def trunc_body(x_ref, o_ref, sems):
        def mk(k):
            return pltpu.make_async_copy(
                x_ref.at[pl.ds(k * nb, nb), pl.ds(0, seq), pl.ds(0, dim)],
                o_ref.at[pl.ds(k * nb, nb)], sems.at[k])

        for k in range(ncopy):
            mk(k).start()
        for k in range(ncopy):
            mk(k).wait()

    return pl.pallas_call(
        trunc_body,
        in_specs=[pl.BlockSpec(memory_space=pltpu.MemorySpace.HBM)],
        out_specs=pl.BlockSpec(memory_space=pltpu.MemorySpace.HBM),
        out_shape=jax.ShapeDtypeStruct((n_batch, seq, dim), jnp.float32),
        scratch_shapes=[pltpu.SemaphoreType.DMA((ncopy,))],
    )


def _make_trunc_v(n_batch, seq, seq_pad, dim, dim_pad, bb):
    def body(x_ref, o_ref):
        o_ref[...] = x_ref[:, :seq, :dim]

    return pl.pallas_call(
        body,
        grid=(n_batch // bb,),
        in_specs=[pl.BlockSpec((bb, seq_pad, dim_pad), lambda i: (i, 0, 0))],
        out_specs=pl.BlockSpec((bb, seq, dim), lambda i: (i, 0, 0)),
        out_shape=jax.ShapeDtypeStruct((n_batch, seq, dim), jnp.float32),
    )


def kernel(idx, table):
    b, t = idx.shape
    vocab, dim = table.shape
    seq_pad = 56
    dim_pad = 1024
    n_workers = 32
    nbuf = 2
    idx_i32 = idx.astype(jnp.int32)
    idx_pad = jnp.concatenate(
        [idx_i32, jnp.broadcast_to(idx_i32[:, -1:], (b, seq_pad - t))],
        axis=1).reshape(b * seq_pad)
    table_pad = jnp.pad(table, ((0, 0), (0, dim_pad - dim)))
    fn = _make_gather(b, seq_pad, dim_pad, n_workers, nbuf)
    x = fn(table_pad, idx_pad)
    trunc = _make_trunc_v(b, t, seq_pad, dim, dim_pad, 4)
    return trunc(x)


# speedup vs baseline: 1.8820x; 1.8820x over previous
import functools

import jax
import jax.numpy as jnp
from jax import lax
from jax.experimental import pallas as pl
from jax.experimental.pallas import tpu as pltpu
from jax.experimental.pallas import tpu_sc as plsc


def _make_gather(n_batch: int, seq_pad: int, dim_pad: int, n_workers: int,
                 nbuf: int):
    batch_per_w = n_batch // n_workers
    idx_per_w = batch_per_w * seq_pad
    mesh = plsc.VectorSubcoreMesh(core_axis_name="c", subcore_axis_name="s")
    num_cores = mesh.num_cores

    @functools.partial(
        pl.kernel,
        out_type=jax.ShapeDtypeStruct((n_batch, seq_pad, dim_pad),
                                      jnp.float32),
        mesh=mesh,
        scratch_types=[
            pltpu.VMEM((idx_per_w,), jnp.int32),
            pltpu.VMEM((nbuf, seq_pad, dim_pad), jnp.float32),
            pltpu.SemaphoreType.DMA,
            pltpu.SemaphoreType.DMA,
        ],
    )
    def gather_kernel(table_hbm, idx_hbm, x_hbm, idx_v, rows_v, gsem, ssem):
        wid = lax.axis_index("s") * num_cores + lax.axis_index("c")
        wb = wid * batch_per_w
        pltpu.sync_copy(idx_hbm.at[pl.ds(wid * idx_per_w, idx_per_w)], idx_v)

        def start_gather(g):
            pltpu.async_copy(
                table_hbm.at[idx_v.at[pl.ds(g * seq_pad, seq_pad)]],
                rows_v.at[g % nbuf], gsem)

        def wait_gather(g):
            pltpu.make_async_copy(
                table_hbm.at[idx_v.at[pl.ds(g * seq_pad, seq_pad)]],
                rows_v.at[g % nbuf], gsem).wait()

        def start_store(g):
            pltpu.async_copy(rows_v.at[g % nbuf], x_hbm.at[wb + g], ssem)

        def wait_store(g):
            pltpu.make_async_copy(rows_v.at[g % nbuf], x_hbm.at[wb + g],
                                  ssem).wait()

        for b in range(nbuf - 1):
            start_gather(b)

        def body(g, _):
            @pl.when(g > 0)
            def _():
                wait_store(g - 1)

            @pl.when(g + nbuf - 1 < batch_per_w)
            def _():
                start_gather(g + nbuf - 1)

            wait_gather(g)
            start_store(g)
            return 0

        lax.fori_loop(0, batch_per_w, body, 0)
        wait_store(batch_per_w - 1)

    return gather_kernel


def kernel(idx, table):
    b, t = idx.shape
    vocab, dim = table.shape
    seq_pad = 56
    dim_pad = 1024
    n_workers = 32
    nbuf = 2
    idx_i32 = idx.astype(jnp.int32)
    idx_pad = jnp.concatenate(
        [idx_i32, jnp.broadcast_to(idx_i32[:, -1:], (b, seq_pad - t))],
        axis=1).reshape(b * seq_pad)
    table_pad = jnp.pad(table, ((0, 0), (0, dim_pad - dim)))
    fn = _make_gather(b, seq_pad, dim_pad, n_workers, nbuf)
    x = fn(table_pad, idx_pad)
    return x[:, :t, :dim]
